# Initial kernel scaffold; baseline (speedup 1.0000x reference)
import jax
import jax.numpy as jnp
from jax.experimental import pallas as pl


def kernel(x, importance, W_proj, b_proj, neuron_emb):
    raise NotImplementedError("write your pallas kernel here")


# speedup vs baseline: 1.1225x; 1.1225x over previous
import jax
import jax.numpy as jnp
from jax import lax
from jax.experimental import pallas as pl

_D_SPACE = 64
_N_QK, _N_V, _N_REL, _N_VAL = 256, 128, 64, 32
_K_QK, _K_V, _K_REL, _K_VAL = 64, 32, 16, 3
_ST = 512


def _dense_kernel(x_ref, imp_ref, wp_ref, bp_ref,
                  eqk_ref, ev_ref, erel_ref, eval_ref,
                  wqk_ref, wv_ref, wrel_ref, wval_ref):
    s = pl.program_id(1)
    x = x_ref[0]
    h = jnp.dot(x, wp_ref[...], preferred_element_type=jnp.float32) + bp_ref[...]
    imp = imp_ref[0]
    for e_ref, o_ref in ((eqk_ref, wqk_ref), (ev_ref, wv_ref),
                         (erel_ref, wrel_ref), (eval_ref, wval_ref)):
        e = e_ref[...]
        en = e / (jnp.sqrt(jnp.sum(e * e, axis=1, keepdims=True)) + 1e-12)
        lg = lax.dot_general(h, en, (((1,), (1,)), ((), ())),
                             preferred_element_type=jnp.float32)
        m = jnp.max(lg, axis=1, keepdims=True)
        ex = jnp.exp(lg - m)
        p = ex / jnp.sum(ex, axis=1, keepdims=True)
        contrib = jnp.dot(imp, p, preferred_element_type=jnp.float32)

        @pl.when(s == 0)
        def _(o_ref=o_ref, contrib=contrib):
            o_ref[0] = contrib

        @pl.when(s != 0)
        def _(o_ref=o_ref, contrib=contrib):
            o_ref[0] += contrib


def _col_bcast(row, m):
    ones = jnp.ones((1, m), jnp.float32)
    return lax.dot_general(row, ones, (((0,), (0,)), ((), ())),
                           precision=lax.Precision.HIGHEST,
                           preferred_element_type=jnp.float32)


def _row_sel(w_row, k):
    n = w_row.shape[1]
    wj = _col_bcast(w_row, n)
    wi = jnp.broadcast_to(w_row, (n, n))
    ij = lax.broadcasted_iota(jnp.int32, (n, n), 0)
    ii = lax.broadcasted_iota(jnp.int32, (n, n), 1)
    beats = (wj > wi) | ((wj == wi) & (ij < ii))
    rank = jnp.sum(beats.astype(jnp.float32), axis=0, keepdims=True)
    return rank < float(k)


def _row_sorted_idx(self, k):
    n = self.shape[1]
    ij = lax.broadcasted_iota(jnp.int32, (n, n), 0)
    ii = lax.broadcasted_iota(jnp.int32, (n, n), 1)
    tri = (ij <= ii).astype(jnp.float32)
    csum = jnp.dot(self, tri, precision=lax.Precision.HIGHEST,
                   preferred_element_type=jnp.float32)
    pos = csum - 1.0
    pos_mat = _col_bcast(pos, k)
    sel_mat = _col_bcast(self, k)
    kio = lax.broadcasted_iota(jnp.int32, (n, k), 1).astype(jnp.float32)
    iio = lax.broadcasted_iota(jnp.int32, (n, k), 0).astype(jnp.float32)
    onehot = sel_mat * (pos_mat == kio).astype(jnp.float32)
    idx = jnp.sum(onehot * iio, axis=0, keepdims=True)
    return idx.astype(jnp.int32)


def _route_kernel(wqk_ref, wv_ref, wrel_ref, wval_ref,
                  iqk_ref, iv_ref, rw_ref, vw_ref):
    B = wqk_ref.shape[0]
    for b in range(B):
        for w_ref, k, i_ref in ((wqk_ref, _K_QK, iqk_ref), (wv_ref, _K_V, iv_ref)):
            w_row = w_ref[b:b + 1, :]
            sel = _row_sel(w_row, k).astype(jnp.float32)
            i_ref[b:b + 1, :] = _row_sorted_idx(sel, k)
        for w_ref, k, o_ref in ((wrel_ref, _K_REL, rw_ref), (wval_ref, _K_VAL, vw_ref)):
            w_row = w_ref[b:b + 1, :]
            sw = w_row * _row_sel(w_row, k).astype(jnp.float32)
            o_ref[b:b + 1, :] = sw / (jnp.sum(sw, axis=1, keepdims=True) + 1e-8)


def kernel(x, importance, W_proj, b_proj, neuron_emb):
    B, S, D = x.shape
    emb_qk = neuron_emb[:_N_QK]
    emb_v = neuron_emb[_N_QK:_N_QK + _N_V]
    emb_rel = neuron_emb[_N_QK + _N_V:_N_QK + _N_V + _N_REL]
    emb_val = neuron_emb[_N_QK + _N_V + _N_REL:_N_QK + _N_V + _N_REL + _N_VAL]
    bp = b_proj.reshape(1, _D_SPACE)
    ns = S // _ST

    sizes = (_N_QK, _N_V, _N_REL, _N_VAL)
    imp3 = importance.reshape(B, 1, S)
    wqk, wv, wrel, wval = pl.pallas_call(
        _dense_kernel,
        grid=(B, ns),
        in_specs=[
            pl.BlockSpec((1, _ST, D), lambda b, s: (b, s, 0)),
            pl.BlockSpec((1, 1, _ST), lambda b, s: (b, 0, s)),
            pl.BlockSpec((D, _D_SPACE), lambda b, s: (0, 0)),
            pl.BlockSpec((1, _D_SPACE), lambda b, s: (0, 0)),
            pl.BlockSpec((_N_QK, _D_SPACE), lambda b, s: (0, 0)),
            pl.BlockSpec((_N_V, _D_SPACE), lambda b, s: (0, 0)),
            pl.BlockSpec((_N_REL, _D_SPACE), lambda b, s: (0, 0)),
            pl.BlockSpec((_N_VAL, _D_SPACE), lambda b, s: (0, 0)),
        ],
        out_specs=[pl.BlockSpec((1, 1, n), lambda b, s: (b, 0, 0)) for n in sizes],
        out_shape=[jax.ShapeDtypeStruct((B, 1, n), jnp.float32) for n in sizes],
    )(x, imp3, W_proj, bp, emb_qk, emb_v, emb_rel, emb_val)
    wqk, wv, wrel, wval = (a.reshape(B, n) for a, n in zip((wqk, wv, wrel, wval), sizes))

    iqk, iv, rw, vw = pl.pallas_call(
        _route_kernel,
        out_shape=[
            jax.ShapeDtypeStruct((B, _K_QK), jnp.int32),
            jax.ShapeDtypeStruct((B, _K_V), jnp.int32),
            jax.ShapeDtypeStruct((B, _N_REL), jnp.float32),
            jax.ShapeDtypeStruct((B, _N_VAL), jnp.float32),
        ],
    )(wqk, wv, wrel, wval)

    return (iqk, iv, rw, rw, vw)


# speedup vs baseline: 1.1238x; 1.0011x over previous
import jax
import jax.numpy as jnp
from jax import lax
from jax.experimental import pallas as pl
from jax.experimental.pallas import tpu as pltpu

_D_SPACE = 64
_N_QK, _N_V, _N_REL, _N_VAL = 256, 128, 64, 32
_K_QK, _K_V, _K_REL, _K_VAL = 64, 32, 16, 3
_ST = 512


def _dense_kernel(x_ref, imp_ref, wp_ref, bp_ref,
                  eqk_ref, ev_ref, erel_ref, eval_ref,
                  wqk_ref, wv_ref, wrel_ref, wval_ref):
    s = pl.program_id(1)
    x = x_ref[0]
    h = jnp.dot(x, wp_ref[...], preferred_element_type=jnp.float32) + bp_ref[...]
    imp = imp_ref[0]
    for e_ref, o_ref in ((eqk_ref, wqk_ref), (ev_ref, wv_ref),
                         (erel_ref, wrel_ref), (eval_ref, wval_ref)):
        e = e_ref[...]
        en = e / (jnp.sqrt(jnp.sum(e * e, axis=1, keepdims=True)) + 1e-12)
        lg = lax.dot_general(h, en, (((1,), (1,)), ((), ())),
                             preferred_element_type=jnp.float32)
        m = jnp.max(lg, axis=1, keepdims=True)
        ex = jnp.exp(lg - m)
        p = ex / jnp.sum(ex, axis=1, keepdims=True)
        contrib = jnp.dot(imp, p, preferred_element_type=jnp.float32)

        @pl.when(s == 0)
        def _(o_ref=o_ref, contrib=contrib):
            o_ref[0] = contrib

        @pl.when(s != 0)
        def _(o_ref=o_ref, contrib=contrib):
            o_ref[0] += contrib


def _col_bcast(row, m):
    ones = jnp.ones((1, m), jnp.float32)
    return lax.dot_general(row, ones, (((0,), (0,)), ((), ())),
                           precision=lax.Precision.HIGHEST,
                           preferred_element_type=jnp.float32)


def _row_sel(w_row, k):
    n = w_row.shape[1]
    wj = _col_bcast(w_row, n)
    wi = jnp.broadcast_to(w_row, (n, n))
    ij = lax.broadcasted_iota(jnp.int32, (n, n), 0)
    ii = lax.broadcasted_iota(jnp.int32, (n, n), 1)
    beats = (wj > wi) | ((wj == wi) & (ij < ii))
    rank = jnp.sum(beats.astype(jnp.float32), axis=0, keepdims=True)
    return rank < float(k)


def _row_sorted_idx(self, k):
    n = self.shape[1]
    ij = lax.broadcasted_iota(jnp.int32, (n, n), 0)
    ii = lax.broadcasted_iota(jnp.int32, (n, n), 1)
    tri = (ij <= ii).astype(jnp.float32)
    csum = jnp.dot(self, tri, precision=lax.Precision.HIGHEST,
                   preferred_element_type=jnp.float32)
    pos = csum - 1.0
    pos_mat = _col_bcast(pos, k)
    sel_mat = _col_bcast(self, k)
    kio = lax.broadcasted_iota(jnp.int32, (n, k), 1).astype(jnp.float32)
    iio = lax.broadcasted_iota(jnp.int32, (n, k), 0).astype(jnp.float32)
    onehot = sel_mat * (pos_mat == kio).astype(jnp.float32)
    idx = jnp.sum(onehot * iio, axis=0, keepdims=True)
    return idx.astype(jnp.int32)


def _route_kernel(wqk_ref, wv_ref, wrel_ref, wval_ref,
                  iqk_ref, iv_ref, rw_ref, vw_ref):
    B = wqk_ref.shape[0]
    for b in range(B):
        for w_ref, k, i_ref in ((wqk_ref, _K_QK, iqk_ref), (wv_ref, _K_V, iv_ref)):
            w_row = w_ref[b:b + 1, :]
            sel = _row_sel(w_row, k).astype(jnp.float32)
            i_ref[b:b + 1, :] = _row_sorted_idx(sel, k)
        for w_ref, k, o_ref in ((wrel_ref, _K_REL, rw_ref), (wval_ref, _K_VAL, vw_ref)):
            w_row = w_ref[b:b + 1, :]
            sw = w_row * _row_sel(w_row, k).astype(jnp.float32)
            o_ref[b:b + 1, :] = sw / (jnp.sum(sw, axis=1, keepdims=True) + 1e-8)


def kernel(x, importance, W_proj, b_proj, neuron_emb):
    B, S, D = x.shape
    emb_qk = neuron_emb[:_N_QK]
    emb_v = neuron_emb[_N_QK:_N_QK + _N_V]
    emb_rel = neuron_emb[_N_QK + _N_V:_N_QK + _N_V + _N_REL]
    emb_val = neuron_emb[_N_QK + _N_V + _N_REL:_N_QK + _N_V + _N_REL + _N_VAL]
    bp = b_proj.reshape(1, _D_SPACE)
    ns = S // _ST

    sizes = (_N_QK, _N_V, _N_REL, _N_VAL)
    imp3 = importance.reshape(B, 1, S)
    wqk, wv, wrel, wval = pl.pallas_call(
        _dense_kernel,
        grid=(B, ns),
        in_specs=[
            pl.BlockSpec((1, _ST, D), lambda b, s: (b, s, 0)),
            pl.BlockSpec((1, 1, _ST), lambda b, s: (b, 0, s)),
            pl.BlockSpec((D, _D_SPACE), lambda b, s: (0, 0)),
            pl.BlockSpec((1, _D_SPACE), lambda b, s: (0, 0)),
            pl.BlockSpec((_N_QK, _D_SPACE), lambda b, s: (0, 0)),
            pl.BlockSpec((_N_V, _D_SPACE), lambda b, s: (0, 0)),
            pl.BlockSpec((_N_REL, _D_SPACE), lambda b, s: (0, 0)),
            pl.BlockSpec((_N_VAL, _D_SPACE), lambda b, s: (0, 0)),
        ],
        out_specs=[pl.BlockSpec((1, 1, n), lambda b, s: (b, 0, 0)) for n in sizes],
        out_shape=[jax.ShapeDtypeStruct((B, 1, n), jnp.float32) for n in sizes],
        compiler_params=pltpu.CompilerParams(
            dimension_semantics=("parallel", "arbitrary")),
    )(x, imp3, W_proj, bp, emb_qk, emb_v, emb_rel, emb_val)
    wqk, wv, wrel, wval = (a.reshape(B, n) for a, n in zip((wqk, wv, wrel, wval), sizes))

    iqk, iv, rw, vw = pl.pallas_call(
        _route_kernel,
        out_shape=[
            jax.ShapeDtypeStruct((B, _K_QK), jnp.int32),
            jax.ShapeDtypeStruct((B, _K_V), jnp.int32),
            jax.ShapeDtypeStruct((B, _N_REL), jnp.float32),
            jax.ShapeDtypeStruct((B, _N_VAL), jnp.float32),
        ],
    )(wqk, wv, wrel, wval)

    return (iqk, iv, rw, rw, vw)
